# baseline (device time: 30303 ns/iter reference)
import jax
import jax.numpy as jnp
from jax import lax
from jax.experimental import pallas as pl
from jax.experimental.pallas import tpu as pltpu

E_LOCAL = 2


def kernel(x, assign, W1, W2):
    tok, d = x.shape
    assign2 = assign.reshape(tok, 1)

    def body(x_ref, a_ref, w1_ref, w2_ref, out_ref,
             xsend, xrecv, asend, arecv, rsend, rrecv, sems):
        my_x = lax.axis_index("x")
        my_y = lax.axis_index("y")
        peer = (my_x, 1 - my_y)

        barrier = pltpu.get_barrier_semaphore()
        pl.semaphore_signal(barrier, inc=1, device_id=peer,
                            device_id_type=pl.DeviceIdType.MESH)
        pl.semaphore_wait(barrier, 1)

        xsend[...] = x_ref[...].astype(jnp.bfloat16)
        asend[...] = a_ref[...]

        rdma_x = pltpu.make_async_remote_copy(
            src_ref=xsend, dst_ref=xrecv,
            send_sem=sems.at[0], recv_sem=sems.at[1],
            device_id=peer, device_id_type=pl.DeviceIdType.MESH)
        rdma_a = pltpu.make_async_remote_copy(
            src_ref=asend, dst_ref=arecv,
            send_sem=sems.at[2], recv_sem=sems.at[3],
            device_id=peer, device_id_type=pl.DeviceIdType.MESH)
        rdma_x.start()
        rdma_a.start()

        w1b = [w1_ref[e, :, :].astype(jnp.bfloat16) for e in range(E_LOCAL)]
        w2b = [w2_ref[e, :, :].astype(jnp.bfloat16) for e in range(E_LOCAL)]

        def moe_local_experts(xb, a):
            acc = jnp.zeros((tok, d), jnp.float32)
            for e in range(E_LOCAL):
                ge = my_y * E_LOCAL + e
                h = jnp.maximum(
                    jnp.dot(xb, w1b[e], preferred_element_type=jnp.float32),
                    0.0)
                y = jnp.dot(h.astype(jnp.bfloat16), w2b[e],
                            preferred_element_type=jnp.float32)
                acc = acc + jnp.where(a == ge, y, 0.0)
            return acc

        acc_mine = moe_local_experts(xsend[...], a_ref[...])

        rdma_x.wait()
        rdma_a.wait()

        rsend[...] = moe_local_experts(
            xrecv[...], arecv[...]).astype(jnp.bfloat16)

        rdma_r = pltpu.make_async_remote_copy(
            src_ref=rsend, dst_ref=rrecv,
            send_sem=sems.at[4], recv_sem=sems.at[5],
            device_id=peer, device_id_type=pl.DeviceIdType.MESH)
        rdma_r.start()
        rdma_r.wait()

        out_ref[...] = acc_mine + rrecv[...].astype(jnp.float32)

    return pl.pallas_call(
        body,
        out_shape=jax.ShapeDtypeStruct((tok, d), jnp.float32),
        in_specs=[pl.BlockSpec(memory_space=pltpu.VMEM)] * 4,
        out_specs=pl.BlockSpec(memory_space=pltpu.VMEM),
        scratch_shapes=[
            pltpu.VMEM((tok, d), jnp.bfloat16),
            pltpu.VMEM((tok, d), jnp.bfloat16),
            pltpu.VMEM((tok, 1), jnp.int32),
            pltpu.VMEM((tok, 1), jnp.int32),
            pltpu.VMEM((tok, d), jnp.bfloat16),
            pltpu.VMEM((tok, d), jnp.bfloat16),
            pltpu.SemaphoreType.DMA((6,)),
        ],
        compiler_params=pltpu.CompilerParams(collective_id=0),
    )(x, assign2, W1, W2)


# device time: 27049 ns/iter; 1.1203x vs baseline; 1.1203x over previous
import jax
import jax.numpy as jnp
from jax import lax
from jax.experimental import pallas as pl
from jax.experimental.pallas import tpu as pltpu

E_LOCAL = 2


def kernel(x, assign, W1, W2):
    tok, d = x.shape
    half = tok // 2
    assign2 = assign.reshape(tok, 1)

    def body(x_ref, a_ref, w1_ref, w2_ref, out_ref,
             xsend, xrecv, asend, arecv, rsend, rrecv, osend, orecv, sems):
        my_x = lax.axis_index("x")
        my_y = lax.axis_index("y")
        ypeer = (my_x, 1 - my_y)
        xpeer = (1 - my_x, my_y)

        barrier = pltpu.get_barrier_semaphore()
        for nbr in (ypeer, xpeer):
            pl.semaphore_signal(barrier, inc=1, device_id=nbr,
                                device_id_type=pl.DeviceIdType.MESH)
        pl.semaphore_wait(barrier, 2)

        rows = pl.ds(my_x * half, half)
        xsend[...] = x_ref[rows, :].astype(jnp.bfloat16)
        asend[...] = a_ref[rows, :]

        rdma_x = pltpu.make_async_remote_copy(
            src_ref=xsend, dst_ref=xrecv,
            send_sem=sems.at[0], recv_sem=sems.at[1],
            device_id=ypeer, device_id_type=pl.DeviceIdType.MESH)
        rdma_a = pltpu.make_async_remote_copy(
            src_ref=asend, dst_ref=arecv,
            send_sem=sems.at[2], recv_sem=sems.at[3],
            device_id=ypeer, device_id_type=pl.DeviceIdType.MESH)
        rdma_x.start()
        rdma_a.start()

        w1b = [w1_ref[e, :, :].astype(jnp.bfloat16) for e in range(E_LOCAL)]
        w2b = [w2_ref[e, :, :].astype(jnp.bfloat16) for e in range(E_LOCAL)]

        def moe_local_experts(xb, a):
            acc = jnp.zeros((half, d), jnp.float32)
            for e in range(E_LOCAL):
                ge = my_y * E_LOCAL + e
                h = jnp.maximum(
                    jnp.dot(xb, w1b[e], preferred_element_type=jnp.float32),
                    0.0)
                y = jnp.dot(h.astype(jnp.bfloat16), w2b[e],
                            preferred_element_type=jnp.float32)
                acc = acc + jnp.where(a == ge, y, 0.0)
            return acc

        acc_mine = moe_local_experts(xsend[...], asend[...])

        rdma_x.wait()
        rdma_a.wait()

        rsend[...] = moe_local_experts(
            xrecv[...], arecv[...]).astype(jnp.bfloat16)

        rdma_r = pltpu.make_async_remote_copy(
            src_ref=rsend, dst_ref=rrecv,
            send_sem=sems.at[4], recv_sem=sems.at[5],
            device_id=ypeer, device_id_type=pl.DeviceIdType.MESH)
        rdma_r.start()
        rdma_r.wait()

        myhalf = acc_mine + rrecv[...].astype(jnp.float32)
        osend[...] = myhalf.astype(jnp.bfloat16)
        out_ref[rows, :] = myhalf

        rdma_o = pltpu.make_async_remote_copy(
            src_ref=osend, dst_ref=orecv,
            send_sem=sems.at[6], recv_sem=sems.at[7],
            device_id=xpeer, device_id_type=pl.DeviceIdType.MESH)
        rdma_o.start()
        rdma_o.wait()

        out_ref[pl.ds((1 - my_x) * half, half), :] = (
            orecv[...].astype(jnp.float32))

    return pl.pallas_call(
        body,
        out_shape=jax.ShapeDtypeStruct((tok, d), jnp.float32),
        in_specs=[pl.BlockSpec(memory_space=pltpu.VMEM)] * 4,
        out_specs=pl.BlockSpec(memory_space=pltpu.VMEM),
        scratch_shapes=[
            pltpu.VMEM((half, d), jnp.bfloat16),
            pltpu.VMEM((half, d), jnp.bfloat16),
            pltpu.VMEM((half, 1), jnp.int32),
            pltpu.VMEM((half, 1), jnp.int32),
            pltpu.VMEM((half, d), jnp.bfloat16),
            pltpu.VMEM((half, d), jnp.bfloat16),
            pltpu.VMEM((half, d), jnp.bfloat16),
            pltpu.VMEM((half, d), jnp.bfloat16),
            pltpu.SemaphoreType.DMA((8,)),
        ],
        compiler_params=pltpu.CompilerParams(collective_id=0),
    )(x, assign2, W1, W2)


# device time: 25776 ns/iter; 1.1756x vs baseline; 1.0494x over previous
import jax
import jax.numpy as jnp
from jax import lax
from jax.experimental import pallas as pl
from jax.experimental.pallas import tpu as pltpu

E_LOCAL = 2
N_CHUNK = 2


def kernel(x, assign, W1, W2):
    tok, d = x.shape
    half = tok // 2
    chunk = half // N_CHUNK
    assign2 = assign.reshape(tok, 1)

    def body(x_ref, a_ref, w1_ref, w2_ref, out_ref,
             w1v, w2v, xsend, xrecv, asend, arecv, rsend, rrecv,
             osend, orecv, wsems, sems):
        my_x = lax.axis_index("x")
        my_y = lax.axis_index("y")
        ypeer = (my_x, 1 - my_y)
        xpeer = (1 - my_x, my_y)

        wcopies = []
        for e in range(E_LOCAL):
            c1 = pltpu.make_async_copy(w1_ref.at[e], w1v.at[e], wsems.at[2 * e])
            c2 = pltpu.make_async_copy(w2_ref.at[e], w2v.at[e], wsems.at[2 * e + 1])
            c1.start()
            c2.start()
            wcopies.append((c1, c2))

        barrier = pltpu.get_barrier_semaphore()
        for nbr in (ypeer, xpeer):
            pl.semaphore_signal(barrier, inc=1, device_id=nbr,
                                device_id_type=pl.DeviceIdType.MESH)
        pl.semaphore_wait(barrier, 2)

        rows = pl.ds(my_x * half, half)
        xsend[...] = x_ref[rows, :].astype(jnp.bfloat16)
        asend[...] = a_ref[rows, :]

        rdma_x = pltpu.make_async_remote_copy(
            src_ref=xsend, dst_ref=xrecv,
            send_sem=sems.at[0], recv_sem=sems.at[1],
            device_id=ypeer, device_id_type=pl.DeviceIdType.MESH)
        rdma_a = pltpu.make_async_remote_copy(
            src_ref=asend, dst_ref=arecv,
            send_sem=sems.at[2], recv_sem=sems.at[3],
            device_id=ypeer, device_id_type=pl.DeviceIdType.MESH)
        rdma_x.start()
        rdma_a.start()

        w1b = []
        w2b = []
        for e in range(E_LOCAL):
            c1, c2 = wcopies[e]
            c1.wait()
            c2.wait()
            w1b.append(w1v[e, :, :].astype(jnp.bfloat16))
            w2b.append(w2v[e, :, :].astype(jnp.bfloat16))

        def moe_local_experts(xb, a, m):
            acc = jnp.zeros((m, d), jnp.float32)
            for e in range(E_LOCAL):
                ge = my_y * E_LOCAL + e
                h = jnp.maximum(
                    jnp.dot(xb, w1b[e], preferred_element_type=jnp.float32),
                    0.0)
                y = jnp.dot(h.astype(jnp.bfloat16), w2b[e],
                            preferred_element_type=jnp.float32)
                acc = acc + jnp.where(a == ge, y, 0.0)
            return acc

        acc_mine = moe_local_experts(xsend[...], asend[...], half)

        rdma_x.wait()
        rdma_a.wait()

        rdmas_r = []
        for c in range(N_CHUNK):
            cs = pl.ds(c * chunk, chunk)
            rsend[cs, :] = moe_local_experts(
                xrecv[cs, :], arecv[cs, :], chunk).astype(jnp.bfloat16)
            rdma_r = pltpu.make_async_remote_copy(
                src_ref=rsend.at[cs, :], dst_ref=rrecv.at[cs, :],
                send_sem=sems.at[4 + c], recv_sem=sems.at[4 + N_CHUNK + c],
                device_id=ypeer, device_id_type=pl.DeviceIdType.MESH)
            rdma_r.start()
            rdmas_r.append(rdma_r)

        rdmas_o = []
        for c in range(N_CHUNK):
            cs = pl.ds(c * chunk, chunk)
            rdmas_r[c].wait_recv()
            myout = acc_mine[c * chunk:(c + 1) * chunk, :] + (
                rrecv[cs, :].astype(jnp.float32))
            osend[cs, :] = myout.astype(jnp.bfloat16)
            out_ref[pl.ds(my_x * half + c * chunk, chunk), :] = myout
            rdma_o = pltpu.make_async_remote_copy(
                src_ref=osend.at[cs, :], dst_ref=orecv.at[cs, :],
                send_sem=sems.at[4 + 2 * N_CHUNK + c],
                recv_sem=sems.at[4 + 3 * N_CHUNK + c],
                device_id=xpeer, device_id_type=pl.DeviceIdType.MESH)
            rdma_o.start()
            rdmas_o.append(rdma_o)

        for c in range(N_CHUNK):
            cs = pl.ds(c * chunk, chunk)
            rdmas_o[c].wait_recv()
            out_ref[pl.ds((1 - my_x) * half + c * chunk, chunk), :] = (
                orecv[cs, :].astype(jnp.float32))

        for c in range(N_CHUNK):
            rdmas_r[c].wait_send()
            rdmas_o[c].wait_send()

    return pl.pallas_call(
        body,
        out_shape=jax.ShapeDtypeStruct((tok, d), jnp.float32),
        in_specs=[
            pl.BlockSpec(memory_space=pltpu.VMEM),
            pl.BlockSpec(memory_space=pltpu.VMEM),
            pl.BlockSpec(memory_space=pltpu.MemorySpace.HBM),
            pl.BlockSpec(memory_space=pltpu.MemorySpace.HBM),
        ],
        out_specs=pl.BlockSpec(memory_space=pltpu.VMEM),
        scratch_shapes=[
            pltpu.VMEM(W1.shape, jnp.float32),
            pltpu.VMEM(W2.shape, jnp.float32),
            pltpu.VMEM((half, d), jnp.bfloat16),
            pltpu.VMEM((half, d), jnp.bfloat16),
            pltpu.VMEM((half, 1), jnp.int32),
            pltpu.VMEM((half, 1), jnp.int32),
            pltpu.VMEM((half, d), jnp.bfloat16),
            pltpu.VMEM((half, d), jnp.bfloat16),
            pltpu.VMEM((half, d), jnp.bfloat16),
            pltpu.VMEM((half, d), jnp.bfloat16),
            pltpu.SemaphoreType.DMA((2 * E_LOCAL,)),
            pltpu.SemaphoreType.DMA((4 + 4 * N_CHUNK,)),
        ],
        compiler_params=pltpu.CompilerParams(collective_id=0),
    )(x, assign2, W1, W2)
